# Initial kernel scaffold; baseline (speedup 1.0000x reference)
import jax
import jax.numpy as jnp
from jax.experimental import pallas as pl


def kernel(x, edge_index, batch_index, W_first, b_first, Wc1, bc1, Wt1, bt1, bn1_g, bn1_b, bn1_m, bn1_v, convW, convb, trW, trb, bn_g, bn_b, bn_m, bn_v, W1, b1, W2, b2, W3, b3):
    raise NotImplementedError("write your pallas kernel here")


# speedup vs baseline: 17.7180x; 17.7180x over previous
import functools

import jax
import jax.numpy as jnp
from jax import lax
from jax.experimental import pallas as pl
from jax.experimental.pallas import tpu as pltpu
from jax.experimental.pallas import tpu_sc as plsc

NN = 10000
EE = 320000
NP = 10240
EMB = 64
BB = 128
NC, NS = 2, 16
NW = NC * NS
K = 128
EPW = EE // NW
NCHUNK = -(-EPW // K)
EPW_PAD = NCHUNK * K
EPAD = NW * EPW_PAD
RPT = NP // NS
EMBW = 128


def _deg_body(dstf, wtsf, out, idx_v, w_v, hist):
    c = lax.axis_index("c")
    s = lax.axis_index("s")
    wid = c * NS + s
    pltpu.sync_copy(dstf.at[wid], idx_v)
    pltpu.sync_copy(wtsf.at[wid], w_v)

    zeros16 = jnp.zeros((16,), jnp.float32)

    def zbody(i, carry):
        hist[pl.ds(i * 16, 16)] = zeros16
        return carry

    lax.fori_loop(0, NP // 16, zbody, 0)

    def ebody(t, carry):
        idx = idx_v[pl.ds(t * 16, 16)]
        w = w_v[pl.ds(t * 16, 16)]
        plsc.addupdate_scatter(hist, [idx], w)
        return carry

    lax.fori_loop(0, EPW_PAD // 16, ebody, 0)
    pltpu.sync_copy(hist, out.at[wid])


def _msg_body(g_hbm, z_hbm, srcw, dstw, out, src_v, dst_v, rows_v, acc_s, sem):
    c = lax.axis_index("c")
    s = lax.axis_index("s")
    wid = c * NS + s
    r0 = s * RPT
    pltpu.sync_copy(z_hbm.at[pl.ds(r0, RPT)], acc_s.at[pl.ds(r0, RPT)])
    pltpu.sync_copy(srcw.at[wid], src_v)
    pltpu.sync_copy(dstw.at[wid], dst_v)
    plsc.subcore_barrier()

    def body(j, carry):
        pltpu.async_copy(g_hbm.at[src_v.at[j]], rows_v, sem).wait()
        pltpu.sync_copy(rows_v, acc_s.at[dst_v.at[j]], add=True)
        return carry

    lax.fori_loop(0, NCHUNK, body, 0)
    plsc.subcore_barrier()
    pltpu.sync_copy(acc_s.at[pl.ds(r0, RPT)], out.at[c, pl.ds(r0, RPT)])


@functools.cache
def _sc_kernels():
    mesh = plsc.VectorSubcoreMesh(
        core_axis_name="c", subcore_axis_name="s",
        num_cores=NC, num_subcores=NS)
    deg = pl.kernel(
        _deg_body,
        out_type=jax.ShapeDtypeStruct((NW, NP), jnp.float32),
        mesh=mesh,
        compiler_params=pltpu.CompilerParams(needs_layout_passes=False),
        scratch_types=[
            pltpu.VMEM((EPW_PAD,), jnp.int32),
            pltpu.VMEM((EPW_PAD,), jnp.float32),
            pltpu.VMEM((NP,), jnp.float32),
        ],
    )
    msg = pl.kernel(
        _msg_body,
        out_type=jax.ShapeDtypeStruct((NC, NP, EMBW), jnp.float32),
        mesh=mesh,
        scratch_types=[
            pltpu.VMEM((NCHUNK, K), jnp.int32),
            pltpu.VMEM((NCHUNK, K), jnp.int32),
            pltpu.VMEM((K, EMBW), jnp.float32),
            pltpu.VMEM_SHARED((NP, EMBW), jnp.float32),
            pltpu.SemaphoreType.DMA,
        ],
    )
    return deg, msg


def _first_body(x_ref, wf_ref, bf_ref, wc_ref, parts_ref, mask_ref,
                g_ref, dinv_ref):
    deg = jnp.sum(parts_ref[...], axis=0) + mask_ref[0]
    dinv = jnp.where(deg > 0, lax.rsqrt(jnp.maximum(deg, 1e-12)), 0.0)
    dinv = dinv[:, None]
    h0 = jnp.dot(x_ref[...], wf_ref[...],
                 preferred_element_type=jnp.float32) + bf_ref[...]
    g = jnp.dot(h0, wc_ref[...], preferred_element_type=jnp.float32) * dinv
    g_ref[...] = jnp.pad(g, ((0, 0), (0, EMBW - EMB)))
    dinv_ref[...] = dinv


_tc_first = pl.pallas_call(
    _first_body,
    out_shape=[
        jax.ShapeDtypeStruct((NP, EMBW), jnp.float32),
        jax.ShapeDtypeStruct((NP, 1), jnp.float32),
    ],
)


def _post_conv(p_ref, g_ref, dinv_ref, cb_ref, bng_ref, bnb_ref, bnm_ref,
               bnv_ref, wt_ref, bt_ref):
    dinv = dinv_ref[...]
    s = (p_ref[0] + p_ref[1] + g_ref[...])[:, :EMB]
    h = jnp.maximum(s * dinv + cb_ref[...], 0.0)
    bns = bng_ref[...] * lax.rsqrt(bnv_ref[...] + 1e-5)
    h = (h - bnm_ref[...]) * bns + bnb_ref[...]
    return jnp.maximum(
        jnp.dot(h, wt_ref[...], preferred_element_type=jnp.float32)
        + bt_ref[...], 0.0)


def _mid_body(p_ref, g_ref, dinv_ref, cb_ref, bng_ref, bnb_ref, bnm_ref,
              bnv_ref, wt_ref, bt_ref, wn_ref, gout_ref):
    h = _post_conv(p_ref, g_ref, dinv_ref, cb_ref, bng_ref, bnb_ref,
                   bnm_ref, bnv_ref, wt_ref, bt_ref)
    g = jnp.dot(h, wn_ref[...], preferred_element_type=jnp.float32) * dinv_ref[...]
    gout_ref[...] = jnp.pad(g, ((0, 0), (0, EMBW - EMB)))


_tc_mid = pl.pallas_call(
    _mid_body,
    out_shape=jax.ShapeDtypeStruct((NP, EMBW), jnp.float32),
)


def _fin_body(p_ref, g_ref, dinv_ref, cb_ref, bng_ref, bnb_ref, bnm_ref,
              bnv_ref, wt_ref, bt_ref, bi_ref, w1_ref, b1_ref, w2_ref,
              b2_ref, w3_ref, b3_ref, out_ref):
    h = _post_conv(p_ref, g_ref, dinv_ref, cb_ref, bng_ref, bnb_ref,
                   bnm_ref, bnv_ref, wt_ref, bt_ref)
    onehot = (bi_ref[...] == lax.broadcasted_iota(
        jnp.int32, (NP, BB), 1)).astype(jnp.float32)
    sums = lax.dot_general(onehot, h, (((0,), (0,)), ((), ())),
                           preferred_element_type=jnp.float32)
    cnt = jnp.sum(onehot, axis=0)
    pooled = sums / jnp.maximum(cnt, 1.0)[:, None]
    w1 = w1_ref[...]
    z = jnp.maximum(
        jnp.dot(pooled, w1[:EMB], preferred_element_type=jnp.float32)
        + jnp.dot(pooled, w1[EMB:], preferred_element_type=jnp.float32)
        + b1_ref[...], 0.0)
    z = jnp.maximum(
        jnp.dot(z, w2_ref[...], preferred_element_type=jnp.float32)
        + b2_ref[...], 0.0)
    out_ref[...] = jnp.dot(
        z, w3_ref[...], preferred_element_type=jnp.float32) + b3_ref[...]


_tc_fin = pl.pallas_call(
    _fin_body,
    out_shape=jax.ShapeDtypeStruct((BB, 10), jnp.float32),
)


def kernel(x, edge_index, batch_index, W_first, b_first, Wc1, bc1, Wt1, bt1,
           bn1_g, bn1_b, bn1_m, bn1_v, convW, convb, trW, trb,
           bn_g, bn_b, bn_m, bn_v, W1, b1, W2, b2, W3, b3):
    f32 = jnp.float32
    npad = NP - NN
    epad = EPAD - EE

    pad_ids = NN + (jnp.arange(epad, dtype=jnp.int32) % npad)
    srcp = jnp.concatenate([edge_index[0], pad_ids])
    dstp = jnp.concatenate([edge_index[1], pad_ids])
    wts = jnp.concatenate([jnp.ones((EE,), f32), jnp.zeros((epad,), f32)])
    srcw = srcp.reshape(NW, NCHUNK, K)
    dstw = dstp.reshape(NW, NCHUNK, K)
    dstf = dstp.reshape(NW, EPW_PAD)
    wtsf = wts.reshape(NW, EPW_PAD)

    xp = jnp.pad(x, ((0, npad), (0, 0)))
    mask = jnp.concatenate([jnp.ones((NN,), f32), jnp.zeros((npad,), f32)])
    mask2 = mask[None, :]
    bip = jnp.pad(batch_index, (0, npad), constant_values=-1)[:, None]
    zz = jnp.zeros((NP, EMBW), f32)

    row = lambda v: v[None, :]

    _deg_kernel, _msg_kernel = _sc_kernels()
    parts_deg = _deg_kernel(dstf, wtsf)
    g, dinv = _tc_first(xp, W_first, row(b_first), Wc1, parts_deg, mask2)

    p = _msg_kernel(g, zz, srcw, dstw)
    g = _tc_mid(p, g, dinv, row(bc1), row(bn1_g), row(bn1_b), row(bn1_m),
                row(bn1_v), Wt1, row(bt1), convW[0])
    for i in range(2):
        p = _msg_kernel(g, zz, srcw, dstw)
        g = _tc_mid(p, g, dinv, row(convb[i]), row(bn_g[i]), row(bn_b[i]),
                    row(bn_m[i]), row(bn_v[i]), trW[i], row(trb[i]),
                    convW[i + 1])
    p = _msg_kernel(g, zz, srcw, dstw)
    out = _tc_fin(p, g, dinv, row(convb[2]), row(bn_g[2]), row(bn_b[2]),
                  row(bn_m[2]), row(bn_v[2]), trW[2], row(trb[2]), bip,
                  W1, row(b1), W2, row(b2), W3, row(b3))
    return out


# speedup vs baseline: 22.9643x; 1.2961x over previous
import functools

import jax
import jax.numpy as jnp
from jax import lax
from jax.experimental import pallas as pl
from jax.experimental.pallas import tpu as pltpu
from jax.experimental.pallas import tpu_sc as plsc

NN = 10000
EE = 320000
NP = 10240
EMB = 64
BB = 128
NC, NS = 2, 16
NW = NC * NS
K = 128
EPW = EE // NW
NCHUNK = -(-EPW // K)
EPW_PAD = NCHUNK * K
EPAD = NW * EPW_PAD
RPT = NP // NS
EMBW = 64


def _deg_body(dstf, wtsf, out, idx_v, w_v, hist):
    c = lax.axis_index("c")
    s = lax.axis_index("s")
    wid = c * NS + s
    pltpu.sync_copy(dstf.at[wid], idx_v)
    pltpu.sync_copy(wtsf.at[wid], w_v)

    zeros16 = jnp.zeros((16,), jnp.float32)

    def zbody(i, carry):
        hist[pl.ds(i * 16, 16)] = zeros16
        return carry

    lax.fori_loop(0, NP // 16, zbody, 0)

    def ebody(t, carry):
        idx = idx_v[pl.ds(t * 16, 16)]
        w = w_v[pl.ds(t * 16, 16)]
        plsc.addupdate_scatter(hist, [idx], w)
        return carry

    lax.fori_loop(0, EPW_PAD // 16, ebody, 0)
    pltpu.sync_copy(hist, out.at[wid])


def _msg_body(g_hbm, z_hbm, srcw, dstw, out, src_v, dst_v, rows_v, acc_s, sem):
    c = lax.axis_index("c")
    s = lax.axis_index("s")
    wid = c * NS + s
    r0 = s * RPT
    pltpu.sync_copy(z_hbm.at[pl.ds(r0, RPT)], acc_s.at[pl.ds(r0, RPT)])
    pltpu.sync_copy(srcw.at[wid], src_v)
    pltpu.sync_copy(dstw.at[wid], dst_v)
    plsc.subcore_barrier()

    def body(j, carry):
        pltpu.async_copy(g_hbm.at[src_v.at[j]], rows_v, sem).wait()
        pltpu.sync_copy(rows_v, acc_s.at[dst_v.at[j]], add=True)
        return carry

    lax.fori_loop(0, NCHUNK, body, 0)
    plsc.subcore_barrier()
    pltpu.sync_copy(acc_s.at[pl.ds(r0, RPT)], out.at[c, pl.ds(r0, RPT)])


@functools.cache
def _sc_kernels():
    mesh = plsc.VectorSubcoreMesh(
        core_axis_name="c", subcore_axis_name="s",
        num_cores=NC, num_subcores=NS)
    deg = pl.kernel(
        _deg_body,
        out_type=jax.ShapeDtypeStruct((NW, NP), jnp.float32),
        mesh=mesh,
        compiler_params=pltpu.CompilerParams(needs_layout_passes=False),
        scratch_types=[
            pltpu.VMEM((EPW_PAD,), jnp.int32),
            pltpu.VMEM((EPW_PAD,), jnp.float32),
            pltpu.VMEM((NP,), jnp.float32),
        ],
    )
    msg = pl.kernel(
        _msg_body,
        out_type=jax.ShapeDtypeStruct((NC, NP, EMBW), jnp.float32),
        mesh=mesh,
        compiler_params=pltpu.CompilerParams(use_tc_tiling_on_sc=False),
        scratch_types=[
            pltpu.VMEM((NCHUNK, K), jnp.int32),
            pltpu.VMEM((NCHUNK, K), jnp.int32),
            pltpu.VMEM((K, EMBW), jnp.float32),
            pltpu.VMEM_SHARED((NP, EMBW), jnp.float32),
            pltpu.SemaphoreType.DMA,
        ],
    )
    return deg, msg


def _first_body(x_ref, wf_ref, bf_ref, wc_ref, parts_ref, mask_ref,
                g_ref, dinv_ref):
    deg = jnp.sum(parts_ref[...], axis=0) + mask_ref[0]
    dinv = jnp.where(deg > 0, lax.rsqrt(jnp.maximum(deg, 1e-12)), 0.0)
    dinv = dinv[:, None]
    h0 = jnp.dot(x_ref[...], wf_ref[...],
                 preferred_element_type=jnp.float32) + bf_ref[...]
    g = jnp.dot(h0, wc_ref[...], preferred_element_type=jnp.float32) * dinv
    g_ref[...] = jnp.pad(g, ((0, 0), (0, EMBW - EMB)))
    dinv_ref[...] = dinv


_tc_first = pl.pallas_call(
    _first_body,
    out_shape=[
        jax.ShapeDtypeStruct((NP, EMBW), jnp.float32),
        jax.ShapeDtypeStruct((NP, 1), jnp.float32),
    ],
)


def _post_conv(p_ref, g_ref, dinv_ref, cb_ref, bng_ref, bnb_ref, bnm_ref,
               bnv_ref, wt_ref, bt_ref):
    dinv = dinv_ref[...]
    s = (p_ref[0] + p_ref[1] + g_ref[...])[:, :EMB]
    h = jnp.maximum(s * dinv + cb_ref[...], 0.0)
    bns = bng_ref[...] * lax.rsqrt(bnv_ref[...] + 1e-5)
    h = (h - bnm_ref[...]) * bns + bnb_ref[...]
    return jnp.maximum(
        jnp.dot(h, wt_ref[...], preferred_element_type=jnp.float32)
        + bt_ref[...], 0.0)


def _mid_body(p_ref, g_ref, dinv_ref, cb_ref, bng_ref, bnb_ref, bnm_ref,
              bnv_ref, wt_ref, bt_ref, wn_ref, gout_ref):
    h = _post_conv(p_ref, g_ref, dinv_ref, cb_ref, bng_ref, bnb_ref,
                   bnm_ref, bnv_ref, wt_ref, bt_ref)
    g = jnp.dot(h, wn_ref[...], preferred_element_type=jnp.float32) * dinv_ref[...]
    gout_ref[...] = jnp.pad(g, ((0, 0), (0, EMBW - EMB)))


_tc_mid = pl.pallas_call(
    _mid_body,
    out_shape=jax.ShapeDtypeStruct((NP, EMBW), jnp.float32),
)


def _fin_body(p_ref, g_ref, dinv_ref, cb_ref, bng_ref, bnb_ref, bnm_ref,
              bnv_ref, wt_ref, bt_ref, bi_ref, w1_ref, b1_ref, w2_ref,
              b2_ref, w3_ref, b3_ref, out_ref):
    h = _post_conv(p_ref, g_ref, dinv_ref, cb_ref, bng_ref, bnb_ref,
                   bnm_ref, bnv_ref, wt_ref, bt_ref)
    onehot = (bi_ref[...] == lax.broadcasted_iota(
        jnp.int32, (NP, BB), 1)).astype(jnp.float32)
    sums = lax.dot_general(onehot, h, (((0,), (0,)), ((), ())),
                           preferred_element_type=jnp.float32)
    cnt = jnp.sum(onehot, axis=0)
    pooled = sums / jnp.maximum(cnt, 1.0)[:, None]
    w1 = w1_ref[...]
    z = jnp.maximum(
        jnp.dot(pooled, w1[:EMB], preferred_element_type=jnp.float32)
        + jnp.dot(pooled, w1[EMB:], preferred_element_type=jnp.float32)
        + b1_ref[...], 0.0)
    z = jnp.maximum(
        jnp.dot(z, w2_ref[...], preferred_element_type=jnp.float32)
        + b2_ref[...], 0.0)
    out_ref[...] = jnp.dot(
        z, w3_ref[...], preferred_element_type=jnp.float32) + b3_ref[...]


_tc_fin = pl.pallas_call(
    _fin_body,
    out_shape=jax.ShapeDtypeStruct((BB, 10), jnp.float32),
)


def kernel(x, edge_index, batch_index, W_first, b_first, Wc1, bc1, Wt1, bt1,
           bn1_g, bn1_b, bn1_m, bn1_v, convW, convb, trW, trb,
           bn_g, bn_b, bn_m, bn_v, W1, b1, W2, b2, W3, b3):
    f32 = jnp.float32
    npad = NP - NN
    epad = EPAD - EE

    pad_ids = NN + (jnp.arange(epad, dtype=jnp.int32) % npad)
    srcp = jnp.concatenate([edge_index[0], pad_ids])
    dstp = jnp.concatenate([edge_index[1], pad_ids])
    wts = jnp.concatenate([jnp.ones((EE,), f32), jnp.zeros((epad,), f32)])
    srcw = srcp.reshape(NW, NCHUNK, K)
    dstw = dstp.reshape(NW, NCHUNK, K)
    dstf = dstp.reshape(NW, EPW_PAD)
    wtsf = wts.reshape(NW, EPW_PAD)

    xp = jnp.pad(x, ((0, npad), (0, 0)))
    mask = jnp.concatenate([jnp.ones((NN,), f32), jnp.zeros((npad,), f32)])
    mask2 = mask[None, :]
    bip = jnp.pad(batch_index, (0, npad), constant_values=-1)[:, None]
    zz = jnp.zeros((NP, EMBW), f32)

    row = lambda v: v[None, :]

    _deg_kernel, _msg_kernel = _sc_kernels()
    parts_deg = _deg_kernel(dstf, wtsf)
    g, dinv = _tc_first(xp, W_first, row(b_first), Wc1, parts_deg, mask2)

    p = _msg_kernel(g, zz, srcw, dstw)
    g = _tc_mid(p, g, dinv, row(bc1), row(bn1_g), row(bn1_b), row(bn1_m),
                row(bn1_v), Wt1, row(bt1), convW[0])
    for i in range(2):
        p = _msg_kernel(g, zz, srcw, dstw)
        g = _tc_mid(p, g, dinv, row(convb[i]), row(bn_g[i]), row(bn_b[i]),
                    row(bn_m[i]), row(bn_v[i]), trW[i], row(trb[i]),
                    convW[i + 1])
    p = _msg_kernel(g, zz, srcw, dstw)
    out = _tc_fin(p, g, dinv, row(convb[2]), row(bn_g[2]), row(bn_b[2]),
                  row(bn_m[2]), row(bn_v[2]), trW[2], row(trb[2]), bip,
                  W1, row(b1), W2, row(b2), W3, row(b3))
    return out


# speedup vs baseline: 35.3582x; 1.5397x over previous
import functools

import jax
import jax.numpy as jnp
from jax import lax
from jax.experimental import pallas as pl
from jax.experimental.pallas import tpu as pltpu
from jax.experimental.pallas import tpu_sc as plsc

NN = 10000
EE = 320000
NP = 10240
EMB = 64
BB = 128
NC, NS = 2, 16
NW = NC * NS
K = 128
EPW = EE // NW
G = 4
NCHUNK = 80
NGROUP = NCHUNK // G
EPW_PAD = NCHUNK * K
EPAD = NW * EPW_PAD
RPT = NP // NS
EMBW = 64


def _deg_body(dstf, wtsf, out, idx_v, w_v, hist):
    c = lax.axis_index("c")
    s = lax.axis_index("s")
    wid = c * NS + s
    pltpu.sync_copy(dstf.at[wid], idx_v)
    pltpu.sync_copy(wtsf.at[wid], w_v)

    zeros16 = jnp.zeros((16,), jnp.float32)

    def zbody(i, carry):
        hist[pl.ds(i * 16, 16)] = zeros16
        return carry

    lax.fori_loop(0, NP // 16, zbody, 0)

    def ebody(t, carry):
        idx = idx_v[pl.ds(t * 16, 16)]
        w = w_v[pl.ds(t * 16, 16)]
        plsc.addupdate_scatter(hist, [idx], w)
        return carry

    lax.fori_loop(0, EPW_PAD // 16, ebody, 0)
    pltpu.sync_copy(hist, out.at[wid])


def _msg_body(g_hbm, z_hbm, srcw, dstw, out, src_v, dst_v, rows2, acc_s,
              semg0, semg1, sems0, sems1):
    c = lax.axis_index("c")
    s = lax.axis_index("s")
    wid = c * NS + s
    r0 = s * RPT
    pltpu.sync_copy(z_hbm.at[pl.ds(r0, RPT)], acc_s.at[pl.ds(r0, RPT)])
    pltpu.sync_copy(srcw.at[wid], src_v)
    pltpu.sync_copy(dstw.at[wid], dst_v)
    plsc.subcore_barrier()

    semg = (semg0, semg1)
    sems = (sems0, sems1)

    def issue_gathers(t, s2):
        for b in range(G):
            pltpu.async_copy(
                g_hbm.at[src_v.at[t * G + b]], rows2.at[s2, b], semg[s2])

    issue_gathers(0, 0)
    issue_gathers(1, 1)

    def body(u, carry):
        for s2 in range(2):
            t = 2 * u + s2
            for b in range(G):
                pltpu.make_async_copy(
                    g_hbm.at[src_v.at[t * G + b]], rows2.at[s2, b],
                    semg[s2]).wait()
            for b in range(G):
                pltpu.async_copy(
                    rows2.at[s2, b], acc_s.at[dst_v.at[t * G + b]],
                    sems[s2], add=True)
            for b in range(G):
                pltpu.make_async_copy(
                    rows2.at[s2, b], acc_s.at[dst_v.at[t * G + b]],
                    sems[s2]).wait()
            issue_gathers(jnp.minimum(t + 2, NGROUP - 1), s2)
        return carry

    lax.fori_loop(0, NGROUP // 2, body, 0)
    for s2 in range(2):
        for b in range(G):
            pltpu.make_async_copy(
                g_hbm.at[src_v.at[(NGROUP - 1) * G + b]], rows2.at[s2, b],
                semg[s2]).wait()
    plsc.subcore_barrier()
    pltpu.sync_copy(acc_s.at[pl.ds(r0, RPT)], out.at[c, pl.ds(r0, RPT)])


@functools.cache
def _sc_kernels():
    mesh = plsc.VectorSubcoreMesh(
        core_axis_name="c", subcore_axis_name="s",
        num_cores=NC, num_subcores=NS)
    deg = pl.kernel(
        _deg_body,
        out_type=jax.ShapeDtypeStruct((NW, NP), jnp.float32),
        mesh=mesh,
        compiler_params=pltpu.CompilerParams(needs_layout_passes=False),
        scratch_types=[
            pltpu.VMEM((EPW_PAD,), jnp.int32),
            pltpu.VMEM((EPW_PAD,), jnp.float32),
            pltpu.VMEM((NP,), jnp.float32),
        ],
    )
    msg = pl.kernel(
        _msg_body,
        out_type=jax.ShapeDtypeStruct((NC, NP, EMBW), jnp.float32),
        mesh=mesh,
        compiler_params=pltpu.CompilerParams(use_tc_tiling_on_sc=False),
        scratch_types=[
            pltpu.VMEM((NCHUNK, K), jnp.int32),
            pltpu.VMEM((NCHUNK, K), jnp.int32),
            pltpu.VMEM((2, G, K, EMBW), jnp.float32),
            pltpu.VMEM_SHARED((NP, EMBW), jnp.float32),
            pltpu.SemaphoreType.DMA,
            pltpu.SemaphoreType.DMA,
            pltpu.SemaphoreType.DMA,
            pltpu.SemaphoreType.DMA,
        ],
    )
    return deg, msg


def _first_body(x_ref, wf_ref, bf_ref, wc_ref, parts_ref, mask_ref,
                g_ref, dinv_ref):
    deg = jnp.sum(parts_ref[...], axis=0) + mask_ref[0]
    dinv = jnp.where(deg > 0, lax.rsqrt(jnp.maximum(deg, 1e-12)), 0.0)
    dinv = dinv[:, None]
    h0 = jnp.dot(x_ref[...], wf_ref[...],
                 preferred_element_type=jnp.float32) + bf_ref[...]
    g = jnp.dot(h0, wc_ref[...], preferred_element_type=jnp.float32) * dinv
    g_ref[...] = jnp.pad(g, ((0, 0), (0, EMBW - EMB)))
    dinv_ref[...] = dinv


_tc_first = pl.pallas_call(
    _first_body,
    out_shape=[
        jax.ShapeDtypeStruct((NP, EMBW), jnp.float32),
        jax.ShapeDtypeStruct((NP, 1), jnp.float32),
    ],
)


def _post_conv(p_ref, g_ref, dinv_ref, cb_ref, bng_ref, bnb_ref, bnm_ref,
               bnv_ref, wt_ref, bt_ref):
    dinv = dinv_ref[...]
    s = (p_ref[0] + p_ref[1] + g_ref[...])[:, :EMB]
    h = jnp.maximum(s * dinv + cb_ref[...], 0.0)
    bns = bng_ref[...] * lax.rsqrt(bnv_ref[...] + 1e-5)
    h = (h - bnm_ref[...]) * bns + bnb_ref[...]
    return jnp.maximum(
        jnp.dot(h, wt_ref[...], preferred_element_type=jnp.float32)
        + bt_ref[...], 0.0)


def _mid_body(p_ref, g_ref, dinv_ref, cb_ref, bng_ref, bnb_ref, bnm_ref,
              bnv_ref, wt_ref, bt_ref, wn_ref, gout_ref):
    h = _post_conv(p_ref, g_ref, dinv_ref, cb_ref, bng_ref, bnb_ref,
                   bnm_ref, bnv_ref, wt_ref, bt_ref)
    g = jnp.dot(h, wn_ref[...], preferred_element_type=jnp.float32) * dinv_ref[...]
    gout_ref[...] = jnp.pad(g, ((0, 0), (0, EMBW - EMB)))


_tc_mid = pl.pallas_call(
    _mid_body,
    out_shape=jax.ShapeDtypeStruct((NP, EMBW), jnp.float32),
)


def _fin_body(p_ref, g_ref, dinv_ref, cb_ref, bng_ref, bnb_ref, bnm_ref,
              bnv_ref, wt_ref, bt_ref, bi_ref, w1_ref, b1_ref, w2_ref,
              b2_ref, w3_ref, b3_ref, out_ref):
    h = _post_conv(p_ref, g_ref, dinv_ref, cb_ref, bng_ref, bnb_ref,
                   bnm_ref, bnv_ref, wt_ref, bt_ref)
    onehot = (bi_ref[...] == lax.broadcasted_iota(
        jnp.int32, (NP, BB), 1)).astype(jnp.float32)
    sums = lax.dot_general(onehot, h, (((0,), (0,)), ((), ())),
                           preferred_element_type=jnp.float32)
    cnt = jnp.sum(onehot, axis=0)
    pooled = sums / jnp.maximum(cnt, 1.0)[:, None]
    w1 = w1_ref[...]
    z = jnp.maximum(
        jnp.dot(pooled, w1[:EMB], preferred_element_type=jnp.float32)
        + jnp.dot(pooled, w1[EMB:], preferred_element_type=jnp.float32)
        + b1_ref[...], 0.0)
    z = jnp.maximum(
        jnp.dot(z, w2_ref[...], preferred_element_type=jnp.float32)
        + b2_ref[...], 0.0)
    out_ref[...] = jnp.dot(
        z, w3_ref[...], preferred_element_type=jnp.float32) + b3_ref[...]


_tc_fin = pl.pallas_call(
    _fin_body,
    out_shape=jax.ShapeDtypeStruct((BB, 10), jnp.float32),
)


def kernel(x, edge_index, batch_index, W_first, b_first, Wc1, bc1, Wt1, bt1,
           bn1_g, bn1_b, bn1_m, bn1_v, convW, convb, trW, trb,
           bn_g, bn_b, bn_m, bn_v, W1, b1, W2, b2, W3, b3):
    f32 = jnp.float32
    npad = NP - NN
    epad = EPAD - EE

    pad_ids = NN + (jnp.arange(epad, dtype=jnp.int32) % npad)
    srcp = jnp.concatenate([edge_index[0], pad_ids])
    dstp = jnp.concatenate([edge_index[1], pad_ids])
    wts = jnp.concatenate([jnp.ones((EE,), f32), jnp.zeros((epad,), f32)])
    srcw = srcp.reshape(NW, NCHUNK, K)
    dstw = dstp.reshape(NW, NCHUNK, K)
    dstf = dstp.reshape(NW, EPW_PAD)
    wtsf = wts.reshape(NW, EPW_PAD)

    xp = jnp.pad(x, ((0, npad), (0, 0)))
    mask = jnp.concatenate([jnp.ones((NN,), f32), jnp.zeros((npad,), f32)])
    mask2 = mask[None, :]
    bip = jnp.pad(batch_index, (0, npad), constant_values=-1)[:, None]
    zz = jnp.zeros((NP, EMBW), f32)

    row = lambda v: v[None, :]

    _deg_kernel, _msg_kernel = _sc_kernels()
    parts_deg = _deg_kernel(dstf, wtsf)
    g, dinv = _tc_first(xp, W_first, row(b_first), Wc1, parts_deg, mask2)

    p = _msg_kernel(g, zz, srcw, dstw)
    g = _tc_mid(p, g, dinv, row(bc1), row(bn1_g), row(bn1_b), row(bn1_m),
                row(bn1_v), Wt1, row(bt1), convW[0])
    for i in range(2):
        p = _msg_kernel(g, zz, srcw, dstw)
        g = _tc_mid(p, g, dinv, row(convb[i]), row(bn_g[i]), row(bn_b[i]),
                    row(bn_m[i]), row(bn_v[i]), trW[i], row(trb[i]),
                    convW[i + 1])
    p = _msg_kernel(g, zz, srcw, dstw)
    out = _tc_fin(p, g, dinv, row(convb[2]), row(bn_g[2]), row(bn_b[2]),
                  row(bn_m[2]), row(bn_v[2]), trW[2], row(trb[2]), bip,
                  W1, row(b1), W2, row(b2), W3, row(b3))
    return out


# speedup vs baseline: 41.1315x; 1.1633x over previous
import functools

import jax
import jax.numpy as jnp
from jax import lax
from jax.experimental import pallas as pl
from jax.experimental.pallas import tpu as pltpu
from jax.experimental.pallas import tpu_sc as plsc

NN = 10000
EE = 320000
NP = 10240
EMB = 64
BB = 128
NC, NS = 2, 16
NW = NC * NS
K = 128
EPW = EE // NW
G = 4
NCHUNK = 80
NGROUP = NCHUNK // G
EPW_PAD = NCHUNK * K
EPAD = NW * EPW_PAD
RPT = NP // NS
EMBW = 64
NP2 = NP // 2


def _deg_body(dstf, wtsf, out, idx_v, w_v, hist, hist2):
    c = lax.axis_index("c")
    s = lax.axis_index("s")
    wid = c * NS + s
    pltpu.sync_copy(dstf.at[wid], idx_v)
    pltpu.sync_copy(wtsf.at[wid], w_v)

    zeros16 = jnp.zeros((16,), jnp.float32)

    def zbody(i, carry):
        hist[pl.ds(i * 16, 16)] = zeros16
        return carry

    lax.fori_loop(0, NP // 16, zbody, 0)

    def ebody(t, carry):
        idx = idx_v[pl.ds(t * 16, 16)]
        w = w_v[pl.ds(t * 16, 16)]
        plsc.addupdate_scatter(hist, [idx], w)
        return carry

    lax.fori_loop(0, EPW_PAD // 16, ebody, 0)

    iota16 = lax.iota(jnp.int32, 16)

    def pbody(t, carry):
        base = t * 16
        v0 = plsc.load_gather(hist, [(iota16 + base) * 2])
        hist2[pl.ds(base, 16)] = v0
        v1 = plsc.load_gather(hist, [(iota16 + base) * 2 + 1])
        hist2[pl.ds(NP2 + base, 16)] = v1
        return carry

    lax.fori_loop(0, NP2 // 16, pbody, 0)
    pltpu.sync_copy(hist2, out.at[wid])


def _msg_body(g_hbm, z_hbm, srcw, dstw, out, src_v, dst_v, rows2, acc_s,
              semg0, semg1, sems0, sems1):
    c = lax.axis_index("c")
    s = lax.axis_index("s")
    wid = c * NS + s
    r0 = s * RPT
    pltpu.sync_copy(z_hbm.at[pl.ds(r0, RPT)], acc_s.at[pl.ds(r0, RPT)])
    pltpu.sync_copy(srcw.at[wid], src_v)
    pltpu.sync_copy(dstw.at[wid], dst_v)
    plsc.subcore_barrier()

    semg = (semg0, semg1)
    sems = (sems0, sems1)

    def issue_gathers(t, s2):
        for b in range(G):
            pltpu.async_copy(
                g_hbm.at[src_v.at[t * G + b]], rows2.at[s2, b], semg[s2])

    issue_gathers(0, 0)
    issue_gathers(1, 1)

    def body(u, carry):
        for s2 in range(2):
            t = 2 * u + s2
            for b in range(G):
                pltpu.make_async_copy(
                    g_hbm.at[src_v.at[t * G + b]], rows2.at[s2, b],
                    semg[s2]).wait()
            for b in range(G):
                pltpu.async_copy(
                    rows2.at[s2, b], acc_s.at[dst_v.at[t * G + b]],
                    sems[s2], add=True)
            for b in range(G):
                pltpu.make_async_copy(
                    rows2.at[s2, b], acc_s.at[dst_v.at[t * G + b]],
                    sems[s2]).wait()
            issue_gathers(jnp.minimum(t + 2, NGROUP - 1), s2)
        return carry

    lax.fori_loop(0, NGROUP // 2, body, 0)
    for s2 in range(2):
        for b in range(G):
            pltpu.make_async_copy(
                g_hbm.at[src_v.at[(NGROUP - 1) * G + b]], rows2.at[s2, b],
                semg[s2]).wait()
    plsc.subcore_barrier()
    pltpu.sync_copy(acc_s.at[pl.ds(r0, RPT)], out.at[c, pl.ds(r0, RPT)])


@functools.cache
def _sc_kernels():
    mesh = plsc.VectorSubcoreMesh(
        core_axis_name="c", subcore_axis_name="s",
        num_cores=NC, num_subcores=NS)
    deg = pl.kernel(
        _deg_body,
        out_type=jax.ShapeDtypeStruct((NW, NP), jnp.float32),
        mesh=mesh,
        compiler_params=pltpu.CompilerParams(needs_layout_passes=False),
        scratch_types=[
            pltpu.VMEM((EPW_PAD,), jnp.int32),
            pltpu.VMEM((EPW_PAD,), jnp.float32),
            pltpu.VMEM((NP,), jnp.float32),
            pltpu.VMEM((NP,), jnp.float32),
        ],
    )
    msg = pl.kernel(
        _msg_body,
        out_type=jax.ShapeDtypeStruct((NC, NP, EMBW), jnp.float32),
        mesh=mesh,
        compiler_params=pltpu.CompilerParams(use_tc_tiling_on_sc=False),
        scratch_types=[
            pltpu.VMEM((NCHUNK, K), jnp.int32),
            pltpu.VMEM((NCHUNK, K), jnp.int32),
            pltpu.VMEM((2, G, K, EMBW), jnp.float32),
            pltpu.VMEM_SHARED((NP, EMBW), jnp.float32),
            pltpu.SemaphoreType.DMA,
            pltpu.SemaphoreType.DMA,
            pltpu.SemaphoreType.DMA,
            pltpu.SemaphoreType.DMA,
        ],
    )
    return deg, msg


def _first_body(x_ref, wf_ref, bf_ref, wc_ref, parts_ref, mask_ref,
                g_ref, dinv_ref):
    deg = jnp.sum(parts_ref[...], axis=0) + mask_ref[0]
    dinv = jnp.where(deg > 0, lax.rsqrt(jnp.maximum(deg, 1e-12)), 0.0)
    dinv = dinv[:, None]
    xp = jnp.pad(x_ref[...], ((0, NP - NN), (0, 0)))
    h0 = jnp.dot(xp, wf_ref[...],
                 preferred_element_type=jnp.float32) + bf_ref[...]
    g = jnp.dot(h0, wc_ref[...], preferred_element_type=jnp.float32) * dinv
    g_ref[...] = jnp.concatenate([g[:NP2], g[NP2:]], axis=1)
    dinv_ref[...] = dinv


_tc_first = pl.pallas_call(
    _first_body,
    out_shape=[
        jax.ShapeDtypeStruct((NP2, 2 * EMB), jnp.float32),
        jax.ShapeDtypeStruct((NP, 1), jnp.float32),
    ],
)


def _post_conv(p_ref, g_ref, dinv_ref, cb_ref, bng_ref, bnb_ref, bnm_ref,
               bnv_ref, wt_ref, bt_ref):
    dinv = dinv_ref[...]
    s128 = p_ref[0] + p_ref[1] + g_ref[...]
    s = jnp.concatenate([s128[:, :EMB], s128[:, EMB:]], axis=0)
    h = jnp.maximum(s * dinv + cb_ref[...], 0.0)
    bns = bng_ref[...] * lax.rsqrt(bnv_ref[...] + 1e-5)
    h = (h - bnm_ref[...]) * bns + bnb_ref[...]
    return jnp.maximum(
        jnp.dot(h, wt_ref[...], preferred_element_type=jnp.float32)
        + bt_ref[...], 0.0)


def _mid_body(p_ref, g_ref, dinv_ref, cb_ref, bng_ref, bnb_ref, bnm_ref,
              bnv_ref, wt_ref, bt_ref, wn_ref, gout_ref):
    h = _post_conv(p_ref, g_ref, dinv_ref, cb_ref, bng_ref, bnb_ref,
                   bnm_ref, bnv_ref, wt_ref, bt_ref)
    g = jnp.dot(h, wn_ref[...], preferred_element_type=jnp.float32) * dinv_ref[...]
    gout_ref[...] = jnp.concatenate([g[:NP2], g[NP2:]], axis=1)


_tc_mid = pl.pallas_call(
    _mid_body,
    out_shape=jax.ShapeDtypeStruct((NP2, 2 * EMB), jnp.float32),
)


def _fin_body(p_ref, g_ref, dinv_ref, cb_ref, bng_ref, bnb_ref, bnm_ref,
              bnv_ref, wt_ref, bt_ref, bi_ref, w1_ref, b1_ref, w2_ref,
              b2_ref, w3_ref, b3_ref, out_ref):
    h = _post_conv(p_ref, g_ref, dinv_ref, cb_ref, bng_ref, bnb_ref,
                   bnm_ref, bnv_ref, wt_ref, bt_ref)
    onehot = (bi_ref[...] == lax.broadcasted_iota(
        jnp.int32, (NP, BB), 1)).astype(jnp.float32)
    sums = lax.dot_general(onehot, h, (((0,), (0,)), ((), ())),
                           preferred_element_type=jnp.float32)
    cnt = jnp.sum(onehot, axis=0)
    pooled = sums / jnp.maximum(cnt, 1.0)[:, None]
    w1 = w1_ref[...]
    z = jnp.maximum(
        jnp.dot(pooled, w1[:EMB], preferred_element_type=jnp.float32)
        + jnp.dot(pooled, w1[EMB:], preferred_element_type=jnp.float32)
        + b1_ref[...], 0.0)
    z = jnp.maximum(
        jnp.dot(z, w2_ref[...], preferred_element_type=jnp.float32)
        + b2_ref[...], 0.0)
    out_ref[...] = jnp.dot(
        z, w3_ref[...], preferred_element_type=jnp.float32) + b3_ref[...]


_tc_fin = pl.pallas_call(
    _fin_body,
    out_shape=jax.ShapeDtypeStruct((BB, 10), jnp.float32),
)


def kernel(x, edge_index, batch_index, W_first, b_first, Wc1, bc1, Wt1, bt1,
           bn1_g, bn1_b, bn1_m, bn1_v, convW, convb, trW, trb,
           bn_g, bn_b, bn_m, bn_v, W1, b1, W2, b2, W3, b3):
    f32 = jnp.float32
    npad = NP - NN
    epad = EPAD - EE

    pad_ids = NN + (jnp.arange(epad, dtype=jnp.int32) % npad)
    sc_row = lambda n: 2 * (n % NP2) + n // NP2
    srcp = sc_row(jnp.concatenate([edge_index[0], pad_ids]))
    dstp = sc_row(jnp.concatenate([edge_index[1], pad_ids]))
    wts = jnp.concatenate([jnp.ones((EE,), f32), jnp.zeros((epad,), f32)])
    srcw = srcp.reshape(NW, NCHUNK, K)
    dstw = dstp.reshape(NW, NCHUNK, K)
    dstf = dstp.reshape(NW, EPW_PAD)
    wtsf = wts.reshape(NW, EPW_PAD)

    mask = jnp.concatenate([jnp.ones((NN,), f32), jnp.zeros((npad,), f32)])
    mask2 = mask[None, :]
    bip = jnp.pad(batch_index, (0, npad), constant_values=-1)[:, None]
    zz = jnp.zeros((NP, EMBW), f32)

    row = lambda v: v[None, :]
    sc_view = lambda a: a.reshape(NP, EMB)
    tc_view = lambda a: a.reshape(NC, NP2, 2 * EMB)

    _deg_kernel, _msg_kernel = _sc_kernels()
    parts_deg = _deg_kernel(dstf, wtsf)
    gtc, dinv = _tc_first(x, W_first, row(b_first), Wc1, parts_deg, mask2)

    p = tc_view(_msg_kernel(sc_view(gtc), zz, srcw, dstw))
    gtc = _tc_mid(p, gtc, dinv, row(bc1), row(bn1_g), row(bn1_b), row(bn1_m),
                  row(bn1_v), Wt1, row(bt1), convW[0])
    for i in range(2):
        p = tc_view(_msg_kernel(sc_view(gtc), zz, srcw, dstw))
        gtc = _tc_mid(p, gtc, dinv, row(convb[i]), row(bn_g[i]),
                      row(bn_b[i]), row(bn_m[i]), row(bn_v[i]), trW[i],
                      row(trb[i]), convW[i + 1])
    p = tc_view(_msg_kernel(sc_view(gtc), zz, srcw, dstw))
    out = _tc_fin(p, gtc, dinv, row(convb[2]), row(bn_g[2]), row(bn_b[2]),
                  row(bn_m[2]), row(bn_v[2]), trW[2], row(trb[2]), bip,
                  W1, row(b1), W2, row(b2), W3, row(b3))
    return out


# speedup vs baseline: 43.4717x; 1.0569x over previous
import functools

import jax
import jax.numpy as jnp
from jax import lax
from jax.experimental import pallas as pl
from jax.experimental.pallas import tpu as pltpu
from jax.experimental.pallas import tpu_sc as plsc

NN = 10000
EE = 320000
NP = 10240
EMB = 64
BB = 128
NC, NS = 2, 16
NW = NC * NS
K = 128
EPW = EE // NW
G = 4
NCHUNK = 80
NGROUP = NCHUNK // G
EPW_PAD = NCHUNK * K
EPAD = NW * EPW_PAD
RPT = NP // NS
EMBW = 64
NP2 = NP // 2


def _deg_body(ew, out, idx_v, hist, hist2):
    c = lax.axis_index("c")
    s = lax.axis_index("s")
    wid = c * NS + s
    pltpu.sync_copy(ew.at[1, wid], idx_v)

    zeros16 = jnp.zeros((16,), jnp.float32)
    iota16 = lax.iota(jnp.int32, 16)

    def zbody(i, carry):
        hist[pl.ds(i * 16, 16)] = zeros16
        return carry

    lax.fori_loop(0, NP // 16, zbody, 0)

    def jbody(j, carry):
        for i in range(K // 16):
            idx = idx_v[j, pl.ds(i * 16, 16)]
            pos = j * K + i * 16 + iota16
            w = jnp.where(pos < EPW, 1.0, 0.0).astype(jnp.float32)
            plsc.addupdate_scatter(hist, [idx], w)
        return carry

    lax.fori_loop(0, NCHUNK, jbody, 0)

    iota16 = lax.iota(jnp.int32, 16)

    def pbody(t, carry):
        base = t * 16
        v0 = plsc.load_gather(hist, [(iota16 + base) * 2])
        hist2[pl.ds(base, 16)] = v0
        v1 = plsc.load_gather(hist, [(iota16 + base) * 2 + 1])
        hist2[pl.ds(NP2 + base, 16)] = v1
        return carry

    lax.fori_loop(0, NP2 // 16, pbody, 0)
    pltpu.sync_copy(hist2, out.at[wid])


def _msg_body(g_hbm, z_hbm, ew, out, src_v, dst_v, rows2, acc_s,
              semg0, semg1, sems0, sems1):
    c = lax.axis_index("c")
    s = lax.axis_index("s")
    wid = c * NS + s
    r0 = s * RPT
    pltpu.sync_copy(z_hbm.at[pl.ds(r0, RPT)], acc_s.at[pl.ds(r0, RPT)])
    pltpu.sync_copy(ew.at[0, wid], src_v)
    pltpu.sync_copy(ew.at[1, wid], dst_v)
    plsc.subcore_barrier()

    semg = (semg0, semg1)
    sems = (sems0, sems1)

    def issue_gathers(t, s2):
        for b in range(G):
            pltpu.async_copy(
                g_hbm.at[src_v.at[t * G + b]], rows2.at[s2, b], semg[s2])

    issue_gathers(0, 0)
    issue_gathers(1, 1)

    def body(u, carry):
        for s2 in range(2):
            t = 2 * u + s2
            for b in range(G):
                pltpu.make_async_copy(
                    g_hbm.at[src_v.at[t * G + b]], rows2.at[s2, b],
                    semg[s2]).wait()
            for b in range(G):
                pltpu.async_copy(
                    rows2.at[s2, b], acc_s.at[dst_v.at[t * G + b]],
                    sems[s2], add=True)
            for b in range(G):
                pltpu.make_async_copy(
                    rows2.at[s2, b], acc_s.at[dst_v.at[t * G + b]],
                    sems[s2]).wait()
            issue_gathers(jnp.minimum(t + 2, NGROUP - 1), s2)
        return carry

    lax.fori_loop(0, NGROUP // 2, body, 0)
    for s2 in range(2):
        for b in range(G):
            pltpu.make_async_copy(
                g_hbm.at[src_v.at[(NGROUP - 1) * G + b]], rows2.at[s2, b],
                semg[s2]).wait()
    plsc.subcore_barrier()
    pltpu.sync_copy(acc_s.at[pl.ds(r0, RPT)], out.at[c, pl.ds(r0, RPT)])


@functools.cache
def _sc_kernels():
    mesh = plsc.VectorSubcoreMesh(
        core_axis_name="c", subcore_axis_name="s",
        num_cores=NC, num_subcores=NS)
    deg = pl.kernel(
        _deg_body,
        out_type=jax.ShapeDtypeStruct((NW, NP), jnp.float32),
        mesh=mesh,
        compiler_params=pltpu.CompilerParams(needs_layout_passes=False),
        scratch_types=[
            pltpu.VMEM((NCHUNK, K), jnp.int32),
            pltpu.VMEM((NP,), jnp.float32),
            pltpu.VMEM((NP,), jnp.float32),
        ],
    )
    msg = pl.kernel(
        _msg_body,
        out_type=jax.ShapeDtypeStruct((NC, NP, EMBW), jnp.float32),
        mesh=mesh,
        compiler_params=pltpu.CompilerParams(use_tc_tiling_on_sc=False),
        scratch_types=[
            pltpu.VMEM((NCHUNK, K), jnp.int32),
            pltpu.VMEM((NCHUNK, K), jnp.int32),
            pltpu.VMEM((2, G, K, EMBW), jnp.float32),
            pltpu.VMEM_SHARED((NP, EMBW), jnp.float32),
            pltpu.SemaphoreType.DMA,
            pltpu.SemaphoreType.DMA,
            pltpu.SemaphoreType.DMA,
            pltpu.SemaphoreType.DMA,
        ],
    )
    return deg, msg


def _first_body(x_ref, wf_ref, bf_ref, wc_ref, parts_ref, mask_ref,
                g_ref, dinv_ref):
    deg = jnp.sum(parts_ref[...], axis=0) + mask_ref[0]
    dinv = jnp.where(deg > 0, lax.rsqrt(jnp.maximum(deg, 1e-12)), 0.0)
    dinv = dinv[:, None]
    xp = jnp.pad(x_ref[...], ((0, NP - NN), (0, 0)))
    h0 = jnp.dot(xp, wf_ref[...],
                 preferred_element_type=jnp.float32) + bf_ref[...]
    g = jnp.dot(h0, wc_ref[...], preferred_element_type=jnp.float32) * dinv
    g_ref[...] = jnp.concatenate([g[:NP2], g[NP2:]], axis=1)
    dinv_ref[...] = dinv


_tc_first = pl.pallas_call(
    _first_body,
    out_shape=[
        jax.ShapeDtypeStruct((NP2, 2 * EMB), jnp.float32),
        jax.ShapeDtypeStruct((NP, 1), jnp.float32),
    ],
)


def _post_conv(p_ref, g_ref, dinv_ref, cb_ref, bng_ref, bnb_ref, bnm_ref,
               bnv_ref, wt_ref, bt_ref):
    dinv = dinv_ref[...]
    s128 = p_ref[0] + p_ref[1] + g_ref[...]
    s = jnp.concatenate([s128[:, :EMB], s128[:, EMB:]], axis=0)
    h = jnp.maximum(s * dinv + cb_ref[...], 0.0)
    bns = bng_ref[...] * lax.rsqrt(bnv_ref[...] + 1e-5)
    h = (h - bnm_ref[...]) * bns + bnb_ref[...]
    return jnp.maximum(
        jnp.dot(h, wt_ref[...], preferred_element_type=jnp.float32)
        + bt_ref[...], 0.0)


def _mid_body(p_ref, g_ref, dinv_ref, cb_ref, bng_ref, bnb_ref, bnm_ref,
              bnv_ref, wt_ref, bt_ref, wn_ref, gout_ref):
    h = _post_conv(p_ref, g_ref, dinv_ref, cb_ref, bng_ref, bnb_ref,
                   bnm_ref, bnv_ref, wt_ref, bt_ref)
    g = jnp.dot(h, wn_ref[...], preferred_element_type=jnp.float32) * dinv_ref[...]
    gout_ref[...] = jnp.concatenate([g[:NP2], g[NP2:]], axis=1)


_tc_mid = pl.pallas_call(
    _mid_body,
    out_shape=jax.ShapeDtypeStruct((NP2, 2 * EMB), jnp.float32),
)


def _fin_body(p_ref, g_ref, dinv_ref, cb_ref, bng_ref, bnb_ref, bnm_ref,
              bnv_ref, wt_ref, bt_ref, bi_ref, w1_ref, b1_ref, w2_ref,
              b2_ref, w3_ref, b3_ref, out_ref):
    h = _post_conv(p_ref, g_ref, dinv_ref, cb_ref, bng_ref, bnb_ref,
                   bnm_ref, bnv_ref, wt_ref, bt_ref)
    onehot = (bi_ref[...] == lax.broadcasted_iota(
        jnp.int32, (NP, BB), 1)).astype(jnp.float32)
    sums = lax.dot_general(onehot, h, (((0,), (0,)), ((), ())),
                           preferred_element_type=jnp.float32)
    cnt = jnp.sum(onehot, axis=0)
    pooled = sums / jnp.maximum(cnt, 1.0)[:, None]
    w1 = w1_ref[...]
    z = jnp.maximum(
        jnp.dot(pooled, w1[:EMB], preferred_element_type=jnp.float32)
        + jnp.dot(pooled, w1[EMB:], preferred_element_type=jnp.float32)
        + b1_ref[...], 0.0)
    z = jnp.maximum(
        jnp.dot(z, w2_ref[...], preferred_element_type=jnp.float32)
        + b2_ref[...], 0.0)
    out_ref[...] = jnp.dot(
        z, w3_ref[...], preferred_element_type=jnp.float32) + b3_ref[...]


_tc_fin = pl.pallas_call(
    _fin_body,
    out_shape=jax.ShapeDtypeStruct((BB, 10), jnp.float32),
)


def kernel(x, edge_index, batch_index, W_first, b_first, Wc1, bc1, Wt1, bt1,
           bn1_g, bn1_b, bn1_m, bn1_v, convW, convb, trW, trb,
           bn_g, bn_b, bn_m, bn_v, W1, b1, W2, b2, W3, b3):
    f32 = jnp.float32
    npad = NP - NN
    epad = EPAD - EE

    pad_ids = NN + (jnp.arange(epad, dtype=jnp.int32) % npad)
    sc_row = lambda n: 2 * (n % NP2) + n // NP2
    padw = jnp.broadcast_to(pad_ids.reshape(1, NW, EPW_PAD - EPW),
                            (2, NW, EPW_PAD - EPW))
    ew = sc_row(jnp.concatenate(
        [edge_index.reshape(2, NW, EPW), padw], axis=2))
    ew = ew.reshape(2, NW, NCHUNK, K)

    mask = jnp.concatenate([jnp.ones((NN,), f32), jnp.zeros((npad,), f32)])
    mask2 = mask[None, :]
    bip = jnp.pad(batch_index, (0, npad), constant_values=-1)[:, None]
    zz = jnp.zeros((NP, EMBW), f32)

    row = lambda v: v[None, :]
    sc_view = lambda a: a.reshape(NP, EMB)
    tc_view = lambda a: a.reshape(NC, NP2, 2 * EMB)

    _deg_kernel, _msg_kernel = _sc_kernels()
    parts_deg = _deg_kernel(ew)
    gtc, dinv = _tc_first(x, W_first, row(b_first), Wc1, parts_deg, mask2)

    p = tc_view(_msg_kernel(sc_view(gtc), zz, ew))
    gtc = _tc_mid(p, gtc, dinv, row(bc1), row(bn1_g), row(bn1_b), row(bn1_m),
                  row(bn1_v), Wt1, row(bt1), convW[0])
    for i in range(2):
        p = tc_view(_msg_kernel(sc_view(gtc), zz, ew))
        gtc = _tc_mid(p, gtc, dinv, row(convb[i]), row(bn_g[i]),
                      row(bn_b[i]), row(bn_m[i]), row(bn_v[i]), trW[i],
                      row(trb[i]), convW[i + 1])
    p = tc_view(_msg_kernel(sc_view(gtc), zz, ew))
    out = _tc_fin(p, gtc, dinv, row(convb[2]), row(bn_g[2]), row(bn_b[2]),
                  row(bn_m[2]), row(bn_v[2]), trW[2], row(trb[2]), bip,
                  W1, row(b1), W2, row(b2), W3, row(b3))
    return out


# speedup vs baseline: 44.2717x; 1.0184x over previous
import functools

import jax
import jax.numpy as jnp
from jax import lax
from jax.experimental import pallas as pl
from jax.experimental.pallas import tpu as pltpu
from jax.experimental.pallas import tpu_sc as plsc

NN = 10000
EE = 320000
NP = 10240
EMB = 64
BB = 128
NC, NS = 2, 16
NW = NC * NS
K = 128
EPW = EE // NW
G = 4
NCHUNK = 80
NGROUP = NCHUNK // G
EPW_PAD = NCHUNK * K
EPAD = NW * EPW_PAD
RPT = NP // NS
EMBW = 64
NP2 = NP // 2


def _deg_body(ew, out, idx_v, hist, hist2):
    c = lax.axis_index("c")
    s = lax.axis_index("s")
    wid = c * NS + s
    pltpu.sync_copy(ew.at[1, wid], idx_v)

    zeros16 = jnp.zeros((16,), jnp.float32)
    iota16 = lax.iota(jnp.int32, 16)

    def zbody(i, carry):
        hist[pl.ds(i * 16, 16)] = zeros16
        return carry

    lax.fori_loop(0, NP // 16, zbody, 0)

    def jbody(j, carry):
        for i in range(K // 16):
            idx = idx_v[j, pl.ds(i * 16, 16)]
            pos = j * K + i * 16 + iota16
            w = jnp.where(pos < EPW, 1.0, 0.0).astype(jnp.float32)
            plsc.addupdate_scatter(hist, [idx], w)
        return carry

    lax.fori_loop(0, NCHUNK, jbody, 0)

    iota16 = lax.iota(jnp.int32, 16)

    def pbody(t, carry):
        base = t * 16
        v0 = plsc.load_gather(hist, [(iota16 + base) * 2])
        hist2[pl.ds(base, 16)] = v0
        v1 = plsc.load_gather(hist, [(iota16 + base) * 2 + 1])
        hist2[pl.ds(NP2 + base, 16)] = v1
        return carry

    lax.fori_loop(0, NP2 // 16, pbody, 0)
    pltpu.sync_copy(hist2, out.at[wid])


def _msg_body(g_hbm, z_hbm, ew, out, src_v, dst_v, rows2, acc_s,
              semg0, semg1, sems0, sems1):
    c = lax.axis_index("c")
    s = lax.axis_index("s")
    wid = c * NS + s
    r0 = s * RPT
    dz = pltpu.async_copy(z_hbm.at[pl.ds(r0, RPT)], acc_s.at[pl.ds(r0, RPT)],
                          semg0)
    ds_ = pltpu.async_copy(ew.at[0, wid], src_v, semg1)
    dd = pltpu.async_copy(ew.at[1, wid], dst_v, sems0)
    dz.wait()
    ds_.wait()
    dd.wait()
    plsc.subcore_barrier()

    semg = (semg0, semg1)
    sems = (sems0, sems1)

    def issue_gathers(t, s2):
        for b in range(G):
            pltpu.async_copy(
                g_hbm.at[src_v.at[t * G + b]], rows2.at[s2, b], semg[s2])

    issue_gathers(0, 0)
    issue_gathers(1, 1)

    def body(u, carry):
        for s2 in range(2):
            t = 2 * u + s2
            for b in range(G):
                pltpu.make_async_copy(
                    g_hbm.at[src_v.at[t * G + b]], rows2.at[s2, b],
                    semg[s2]).wait()
            for b in range(G):
                pltpu.async_copy(
                    rows2.at[s2, b], acc_s.at[dst_v.at[t * G + b]],
                    sems[s2], add=True)
            for b in range(G):
                pltpu.make_async_copy(
                    rows2.at[s2, b], acc_s.at[dst_v.at[t * G + b]],
                    sems[s2]).wait()
            issue_gathers(jnp.minimum(t + 2, NGROUP - 1), s2)
        return carry

    lax.fori_loop(0, NGROUP // 2, body, 0)
    for s2 in range(2):
        for b in range(G):
            pltpu.make_async_copy(
                g_hbm.at[src_v.at[(NGROUP - 1) * G + b]], rows2.at[s2, b],
                semg[s2]).wait()
    plsc.subcore_barrier()
    pltpu.sync_copy(acc_s.at[pl.ds(r0, RPT)], out.at[c, pl.ds(r0, RPT)])


@functools.cache
def _sc_kernels():
    mesh = plsc.VectorSubcoreMesh(
        core_axis_name="c", subcore_axis_name="s",
        num_cores=NC, num_subcores=NS)
    deg = pl.kernel(
        _deg_body,
        out_type=jax.ShapeDtypeStruct((NW, NP), jnp.float32),
        mesh=mesh,
        compiler_params=pltpu.CompilerParams(needs_layout_passes=False),
        scratch_types=[
            pltpu.VMEM((NCHUNK, K), jnp.int32),
            pltpu.VMEM((NP,), jnp.float32),
            pltpu.VMEM((NP,), jnp.float32),
        ],
    )
    msg = pl.kernel(
        _msg_body,
        out_type=jax.ShapeDtypeStruct((NC, NP, EMBW), jnp.float32),
        mesh=mesh,
        compiler_params=pltpu.CompilerParams(use_tc_tiling_on_sc=False),
        scratch_types=[
            pltpu.VMEM((NCHUNK, K), jnp.int32),
            pltpu.VMEM((NCHUNK, K), jnp.int32),
            pltpu.VMEM((2, G, K, EMBW), jnp.float32),
            pltpu.VMEM_SHARED((NP, EMBW), jnp.float32),
            pltpu.SemaphoreType.DMA,
            pltpu.SemaphoreType.DMA,
            pltpu.SemaphoreType.DMA,
            pltpu.SemaphoreType.DMA,
        ],
    )
    return deg, msg


def _first_body(x_ref, wf_ref, bf_ref, wc_ref, parts_ref, mask_ref,
                g_ref, dinv_ref):
    deg = jnp.sum(parts_ref[...], axis=0) + mask_ref[0]
    dinv = jnp.where(deg > 0, lax.rsqrt(jnp.maximum(deg, 1e-12)), 0.0)
    dinv = dinv[:, None]
    xp = jnp.pad(x_ref[...], ((0, NP - NN), (0, 0)))
    h0 = jnp.dot(xp, wf_ref[...],
                 preferred_element_type=jnp.float32) + bf_ref[...]
    g = jnp.dot(h0, wc_ref[...], preferred_element_type=jnp.float32) * dinv
    g_ref[...] = jnp.concatenate([g[:NP2], g[NP2:]], axis=1)
    dinv_ref[...] = dinv


_tc_first = pl.pallas_call(
    _first_body,
    out_shape=[
        jax.ShapeDtypeStruct((NP2, 2 * EMB), jnp.float32),
        jax.ShapeDtypeStruct((NP, 1), jnp.float32),
    ],
)


def _post_conv(p_ref, g_ref, dinv_ref, cb_ref, bng_ref, bnb_ref, bnm_ref,
               bnv_ref, wt_ref, bt_ref):
    dinv = dinv_ref[...]
    s128 = p_ref[0] + p_ref[1] + g_ref[...]
    s = jnp.concatenate([s128[:, :EMB], s128[:, EMB:]], axis=0)
    h = jnp.maximum(s * dinv + cb_ref[...], 0.0)
    bns = bng_ref[...] * lax.rsqrt(bnv_ref[...] + 1e-5)
    h = (h - bnm_ref[...]) * bns + bnb_ref[...]
    return jnp.maximum(
        jnp.dot(h, wt_ref[...], preferred_element_type=jnp.float32)
        + bt_ref[...], 0.0)


def _mid_body(p_ref, g_ref, dinv_ref, cb_ref, bng_ref, bnb_ref, bnm_ref,
              bnv_ref, wt_ref, bt_ref, wn_ref, gout_ref):
    h = _post_conv(p_ref, g_ref, dinv_ref, cb_ref, bng_ref, bnb_ref,
                   bnm_ref, bnv_ref, wt_ref, bt_ref)
    g = jnp.dot(h, wn_ref[...], preferred_element_type=jnp.float32) * dinv_ref[...]
    gout_ref[...] = jnp.concatenate([g[:NP2], g[NP2:]], axis=1)


_tc_mid = pl.pallas_call(
    _mid_body,
    out_shape=jax.ShapeDtypeStruct((NP2, 2 * EMB), jnp.float32),
)


def _fin_body(p_ref, g_ref, dinv_ref, cb_ref, bng_ref, bnb_ref, bnm_ref,
              bnv_ref, wt_ref, bt_ref, bi_ref, w1_ref, b1_ref, w2_ref,
              b2_ref, w3_ref, b3_ref, out_ref):
    h = _post_conv(p_ref, g_ref, dinv_ref, cb_ref, bng_ref, bnb_ref,
                   bnm_ref, bnv_ref, wt_ref, bt_ref)
    onehot = (bi_ref[...] == lax.broadcasted_iota(
        jnp.int32, (NP, BB), 1)).astype(jnp.float32)
    sums = lax.dot_general(onehot, h, (((0,), (0,)), ((), ())),
                           preferred_element_type=jnp.float32)
    cnt = jnp.sum(onehot, axis=0)
    pooled = sums / jnp.maximum(cnt, 1.0)[:, None]
    w1 = w1_ref[...]
    z = jnp.maximum(
        jnp.dot(pooled, w1[:EMB], preferred_element_type=jnp.float32)
        + jnp.dot(pooled, w1[EMB:], preferred_element_type=jnp.float32)
        + b1_ref[...], 0.0)
    z = jnp.maximum(
        jnp.dot(z, w2_ref[...], preferred_element_type=jnp.float32)
        + b2_ref[...], 0.0)
    out_ref[...] = jnp.dot(
        z, w3_ref[...], preferred_element_type=jnp.float32) + b3_ref[...]


_tc_fin = pl.pallas_call(
    _fin_body,
    out_shape=jax.ShapeDtypeStruct((BB, 10), jnp.float32),
)


def kernel(x, edge_index, batch_index, W_first, b_first, Wc1, bc1, Wt1, bt1,
           bn1_g, bn1_b, bn1_m, bn1_v, convW, convb, trW, trb,
           bn_g, bn_b, bn_m, bn_v, W1, b1, W2, b2, W3, b3):
    f32 = jnp.float32
    npad = NP - NN
    epad = EPAD - EE

    pad_ids = NN + (jnp.arange(epad, dtype=jnp.int32) % npad)
    sc_row = lambda n: 2 * (n % NP2) + n // NP2
    padw = jnp.broadcast_to(pad_ids.reshape(1, NW, EPW_PAD - EPW),
                            (2, NW, EPW_PAD - EPW))
    ew = sc_row(jnp.concatenate(
        [edge_index.reshape(2, NW, EPW), padw], axis=2))
    ew = ew.reshape(2, NW, NCHUNK, K)

    mask = jnp.concatenate([jnp.ones((NN,), f32), jnp.zeros((npad,), f32)])
    mask2 = mask[None, :]
    bip = jnp.pad(batch_index, (0, npad), constant_values=-1)[:, None]
    zz = jnp.zeros((NP, EMBW), f32)

    row = lambda v: v[None, :]
    sc_view = lambda a: a.reshape(NP, EMB)
    tc_view = lambda a: a.reshape(NC, NP2, 2 * EMB)

    _deg_kernel, _msg_kernel = _sc_kernels()
    parts_deg = _deg_kernel(ew)
    gtc, dinv = _tc_first(x, W_first, row(b_first), Wc1, parts_deg, mask2)

    p = tc_view(_msg_kernel(sc_view(gtc), zz, ew))
    gtc = _tc_mid(p, gtc, dinv, row(bc1), row(bn1_g), row(bn1_b), row(bn1_m),
                  row(bn1_v), Wt1, row(bt1), convW[0])
    for i in range(2):
        p = tc_view(_msg_kernel(sc_view(gtc), zz, ew))
        gtc = _tc_mid(p, gtc, dinv, row(convb[i]), row(bn_g[i]),
                      row(bn_b[i]), row(bn_m[i]), row(bn_v[i]), trW[i],
                      row(trb[i]), convW[i + 1])
    p = tc_view(_msg_kernel(sc_view(gtc), zz, ew))
    out = _tc_fin(p, gtc, dinv, row(convb[2]), row(bn_g[2]), row(bn_b[2]),
                  row(bn_m[2]), row(bn_v[2]), trW[2], row(trb[2]), bip,
                  W1, row(b1), W2, row(b2), W3, row(b3))
    return out
